# baseline (speedup 1.0000x reference)
import functools

import jax
import jax.numpy as jnp
import numpy as np
from jax import lax
from jax.experimental import pallas as pl
from jax.experimental.pallas import tpu as pltpu
from jax.experimental.pallas import tpu_sc as plsc

N = 10000
E = 320000
D = 128
PD = D // 2

NC = 2
NS = 16
NW = NC * NS

NP = 10240
CHUNK = 128
CPT = 80
EPT = CPT * CHUNK
E_PAD = NW * EPT
H = 40

_W = np.arange(D // 2)
LO_COLS = 32 * (_W // 16) + _W % 16

_MESH = plsc.VectorSubcoreMesh(core_axis_name="c", subcore_axis_name="s")


@functools.partial(
    pl.kernel,
    out_type=jax.ShapeDtypeStruct((NC * NP,), jnp.float32),
    mesh=_MESH,
    scratch_types=[
        pltpu.VMEM((CPT, CHUNK), jnp.int32),
        pltpu.VMEM((CHUNK,), jnp.float32),
        pltpu.VMEM_SHARED((NP,), jnp.float32),
    ],
)
def _sc_degree(dst_hbm, zeros_np_hbm, ones_hbm, out_hbm, didx_v, ones_v, acc):
    cid = lax.axis_index("c")
    sid = lax.axis_index("s")
    wid = cid * NS + sid

    @pl.when(sid == 0)
    def _():
        pltpu.sync_copy(zeros_np_hbm, acc)

    pltpu.sync_copy(dst_hbm.at[pl.ds(wid * CPT, CPT)], didx_v)
    pltpu.sync_copy(ones_hbm, ones_v)
    plsc.subcore_barrier()

    def body(t, _):
        pltpu.sync_copy(ones_v, acc.at[didx_v.at[t]], add=True)
        return ()

    lax.fori_loop(0, CPT, body, (), unroll=False)
    plsc.subcore_barrier()

    @pl.when(sid == 0)
    def _():
        pltpu.sync_copy(acc, out_hbm.at[pl.ds(cid * NP, NP)])


@functools.partial(
    pl.kernel,
    out_type=jax.ShapeDtypeStruct((NC * NP, D), jnp.float32),
    mesh=_MESH,
    scratch_types=[
        pltpu.VMEM((H, CHUNK), jnp.int32),
        pltpu.VMEM((H, CHUNK), jnp.int32),
        pltpu.VMEM((CHUNK, PD), jnp.float32),
        pltpu.VMEM((CHUNK, PD), jnp.float32),
        pltpu.VMEM((CHUNK, D), jnp.float32),
        pltpu.VMEM_SHARED((NP, D), jnp.float32),
        pltpu.SemaphoreType.DMA,
        pltpu.SemaphoreType.DMA,
    ],
    compiler_params=pltpu.CompilerParams(
        needs_layout_passes=False, use_tc_tiling_on_sc=False),
)
def _sc_aggregate(g_hbm, src_hbm, dst_hbm, zeros_hbm, out_hbm,
                  sidx_v, didx_v, pk0_v, pk1_v, rows_v, acc, sem0, sem1):
    cid = lax.axis_index("c")
    sid = lax.axis_index("s")
    wid = cid * NS + sid
    rows_per_tile = NP // NS

    pltpu.sync_copy(zeros_hbm, acc.at[pl.ds(sid * rows_per_tile, rows_per_tile)])
    plsc.subcore_barrier()

    bufs = (pk0_v, pk1_v)
    sems = (sem0, sem1)

    def convert(pkb):
        def row(r, _):
            for k in range(4):
                w = pkb[r, pl.ds(16 * k, 16)]
                ab = plsc.bitcast(w, jnp.bfloat16)
                a, b = plsc.unpack(ab, format=plsc.PackFormat.INTERLEAVED)
                rows_v[r, pl.ds(32 * k, 16)] = a
                rows_v[r, pl.ds(32 * k + 16, 16)] = b
            return ()

        lax.fori_loop(0, CHUNK, row, (), unroll=2)

    for phase in range(CPT // H):
        pltpu.sync_copy(src_hbm.at[pl.ds(wid * CPT + phase * H, H)], sidx_v)
        pltpu.sync_copy(dst_hbm.at[pl.ds(wid * CPT + phase * H, H)], didx_v)
        pltpu.async_copy(g_hbm.at[sidx_v.at[0]], pk0_v, sem0)
        pltpu.async_copy(g_hbm.at[sidx_v.at[1]], pk1_v, sem1)

        def body(i, _):
            for b in range(2):
                t = 2 * i + b
                pltpu.make_async_copy(
                    g_hbm.at[sidx_v.at[t]], bufs[b], sems[b]).wait()
                convert(bufs[b])

                @pl.when(t + 2 < H)
                def _():
                    pltpu.async_copy(
                        g_hbm.at[sidx_v.at[t + 2]], bufs[b], sems[b])

                pltpu.sync_copy(rows_v, acc.at[didx_v.at[t]], add=True)
            return ()

        lax.fori_loop(0, H // 2, body, (), unroll=False)
    plsc.subcore_barrier()

    pltpu.sync_copy(
        acc.at[pl.ds(sid * rows_per_tile, rows_per_tile)],
        out_hbm.at[pl.ds(cid * NP + sid * rows_per_tile, rows_per_tile)],
    )


BLK = 512


def _dinv(d0, d1):
    return lax.rsqrt(d0 + d1 + 1.0)


def _hi_mask():
    return jnp.uint32(0xFFFF0000)


def _bf16_bits(g):
    return lax.bitcast_convert_type(
        g.astype(jnp.bfloat16).astype(jnp.float32), jnp.uint32)


def _pack(g_lo, g_hi):
    word = (_bf16_bits(g_hi) & _hi_mask()) | (_bf16_bits(g_lo) >> 16)
    return lax.bitcast_convert_type(word, jnp.float32)


def _unpack_halves(p):
    u = lax.bitcast_convert_type(p, jnp.uint32)
    g_lo = lax.bitcast_convert_type(u << 16, jnp.float32)
    g_hi = lax.bitcast_convert_type(u & _hi_mask(), jnp.float32)
    return g_lo, g_hi


def _scatter_mats():
    w = lax.broadcasted_iota(jnp.int32, (PD, D), 0)
    o = lax.broadcasted_iota(jnp.int32, (PD, D), 1)
    lo_col = 32 * (w // 16) + w % 16
    m_lo = (o == lo_col).astype(jnp.float32)
    m_hi = (o == lo_col + 16).astype(jnp.float32)
    return m_lo, m_hi


def _unpack(p):
    g_lo, g_hi = _unpack_halves(p)
    m_lo, m_hi = _scatter_mats()
    return (jnp.dot(g_lo, m_lo, preferred_element_type=jnp.float32)
            + jnp.dot(g_hi, m_hi, preferred_element_type=jnp.float32))


def _tc_lin_body(x_ref, wlo_ref, whi_ref, d0_ref, d1_ref, o_ref):
    dinv = _dinv(d0_ref[...], d1_ref[...])
    x = x_ref[...]
    g_lo = dinv * jnp.dot(x, wlo_ref[...], preferred_element_type=jnp.float32)
    g_hi = dinv * jnp.dot(x, whi_ref[...], preferred_element_type=jnp.float32)
    o_ref[...] = _pack(g_lo, g_hi)


def _tc_mid_body(p0_ref, p1_ref, g0_ref, d0_ref, d1_ref, b_ref,
                 wlo_ref, whi_ref, o_ref):
    dinv = _dinv(d0_ref[...], d1_ref[...])
    h = dinv * (p0_ref[...] + p1_ref[...] + _unpack(g0_ref[...])) + b_ref[...]
    h = jnp.maximum(h, 0.0)
    nrm = jnp.sqrt(jnp.sum(h * h, axis=1, keepdims=True))
    h = h / jnp.maximum(nrm, 1e-12)
    g_lo = dinv * jnp.dot(h, wlo_ref[...], preferred_element_type=jnp.float32)
    g_hi = dinv * jnp.dot(h, whi_ref[...], preferred_element_type=jnp.float32)
    o_ref[...] = _pack(g_lo, g_hi)


def _tc_fin_body(q0_ref, q1_ref, g1_ref, d0_ref, d1_ref, b_ref, o_ref):
    dinv = _dinv(d0_ref[...], d1_ref[...])
    o_ref[...] = (dinv * (q0_ref[...] + q1_ref[...] + _unpack(g1_ref[...]))
                  + b_ref[...])


def _row_spec(width=D):
    return pl.BlockSpec((BLK, width), lambda i: (i, 0))


def _col_spec():
    return pl.BlockSpec((BLK, 1), lambda i: (i, 0))


def _full_spec(shape):
    return pl.BlockSpec(shape, lambda i: (0,) * len(shape))


_GRID = (NP // BLK,)
_ROW_OUT = jax.ShapeDtypeStruct((NP, D), jnp.float32)
_PK_OUT = jax.ShapeDtypeStruct((NP, PD), jnp.float32)


def _tc_lin(x, wlo, whi, d0, d1):
    return pl.pallas_call(
        _tc_lin_body,
        grid=_GRID,
        in_specs=[_row_spec(), _full_spec((D, PD)), _full_spec((D, PD)),
                  _col_spec(), _col_spec()],
        out_specs=_row_spec(PD),
        out_shape=_PK_OUT,
    )(x, wlo, whi, d0, d1)


def _tc_mid(p0, p1, g0, d0, d1, b, wlo, whi):
    return pl.pallas_call(
        _tc_mid_body,
        grid=_GRID,
        in_specs=[_row_spec(), _row_spec(), _row_spec(PD), _col_spec(),
                  _col_spec(), _full_spec((1, D)), _full_spec((D, PD)),
                  _full_spec((D, PD))],
        out_specs=_row_spec(PD),
        out_shape=_PK_OUT,
    )(p0, p1, g0, d0, d1, b, wlo, whi)


def _tc_fin(q0, q1, g1, d0, d1, b):
    return pl.pallas_call(
        _tc_fin_body,
        grid=_GRID,
        in_specs=[_row_spec(), _row_spec(), _row_spec(PD), _col_spec(),
                  _col_spec(), _full_spec((1, D))],
        out_specs=_row_spec(),
        out_shape=_ROW_OUT,
    )(q0, q1, g1, d0, d1, b)


@jax.jit
def kernel(x, edge_index, W0, b0, W1, b1):
    src = edge_index[0]
    dst = edge_index[1]
    npad = E_PAD - E
    pad_iota = jnp.arange(npad, dtype=jnp.int32)
    pad_dst = N + pad_iota % (NP - N)
    pad_src = pad_iota % N
    src_p = jnp.concatenate([src, pad_src])
    dst_p = jnp.concatenate([dst, pad_dst])
    src2d = src_p.reshape(NW * CPT, CHUNK)
    dst2d = dst_p.reshape(NW * CPT, CHUNK)

    x_pad = jnp.concatenate([x, jnp.zeros((NP - N, D), jnp.float32)])
    zeros_np = jnp.zeros((NP,), jnp.float32)
    ones_chunk = jnp.ones((CHUNK,), jnp.float32)
    zeros_rows = jnp.zeros((NP // NS, D), jnp.float32)

    deg2 = _sc_degree(dst2d, zeros_np, ones_chunk)
    d0 = deg2[:NP].reshape(NP, 1)
    d1 = deg2[NP:].reshape(NP, 1)

    b0_2d = b0.reshape(1, D)
    b1_2d = b1.reshape(1, D)
    w0lo, w0hi = W0[:, LO_COLS], W0[:, LO_COLS + 16]
    w1lo, w1hi = W1[:, LO_COLS], W1[:, LO_COLS + 16]

    g0 = _tc_lin(x_pad, w0lo, w0hi, d0, d1)
    parts0 = _sc_aggregate(g0, src2d, dst2d, zeros_rows)
    g1 = _tc_mid(parts0[:NP], parts0[NP:], g0, d0, d1, b0_2d, w1lo, w1hi)
    parts1 = _sc_aggregate(g1, src2d, dst2d, zeros_rows)
    out = _tc_fin(parts1[:NP], parts1[NP:], g1, d0, d1, b1_2d)
    return out[:N]


# speedup vs baseline: 1.0032x; 1.0032x over previous
import functools

import jax
import jax.numpy as jnp
import numpy as np
from jax import lax
from jax.experimental import pallas as pl
from jax.experimental.pallas import tpu as pltpu
from jax.experimental.pallas import tpu_sc as plsc

N = 10000
E = 320000
D = 128
PD = D // 2

NC = 2
NS = 16
NW = NC * NS

NP = 10240
CHUNK = 128
CPT = 80
EPT = CPT * CHUNK
E_PAD = NW * EPT
H = 40

_W = np.arange(D // 2)
LO_COLS = 32 * (_W // 16) + _W % 16

_MESH = plsc.VectorSubcoreMesh(core_axis_name="c", subcore_axis_name="s")


@functools.partial(
    pl.kernel,
    out_type=jax.ShapeDtypeStruct((NC * NP,), jnp.float32),
    mesh=_MESH,
    scratch_types=[
        pltpu.VMEM((CPT, CHUNK), jnp.int32),
        pltpu.VMEM((CHUNK,), jnp.float32),
        pltpu.VMEM_SHARED((NP,), jnp.float32),
    ],
)
def _sc_degree(dst_hbm, zeros_np_hbm, ones_hbm, out_hbm, didx_v, ones_v, acc):
    cid = lax.axis_index("c")
    sid = lax.axis_index("s")
    wid = cid * NS + sid

    @pl.when(sid == 0)
    def _():
        pltpu.sync_copy(zeros_np_hbm, acc)

    pltpu.sync_copy(dst_hbm.at[pl.ds(wid * CPT, CPT)], didx_v)
    pltpu.sync_copy(ones_hbm, ones_v)
    plsc.subcore_barrier()

    def body(t, _):
        pltpu.sync_copy(ones_v, acc.at[didx_v.at[t]], add=True)
        return ()

    lax.fori_loop(0, CPT, body, (), unroll=False)
    plsc.subcore_barrier()

    @pl.when(sid == 0)
    def _():
        pltpu.sync_copy(acc, out_hbm.at[pl.ds(cid * NP, NP)])


@functools.partial(
    pl.kernel,
    out_type=jax.ShapeDtypeStruct((NC * NP, D), jnp.float32),
    mesh=_MESH,
    scratch_types=[
        pltpu.VMEM((H, CHUNK), jnp.int32),
        pltpu.VMEM((H, CHUNK), jnp.int32),
        pltpu.VMEM((CHUNK, PD), jnp.float32),
        pltpu.VMEM((CHUNK, PD), jnp.float32),
        pltpu.VMEM((CHUNK, D), jnp.float32),
        pltpu.VMEM_SHARED((NP, D), jnp.float32),
        pltpu.SemaphoreType.DMA,
        pltpu.SemaphoreType.DMA,
    ],
    compiler_params=pltpu.CompilerParams(
        needs_layout_passes=False, use_tc_tiling_on_sc=False),
)
def _sc_aggregate(g_hbm, src_hbm, dst_hbm, zeros_hbm, out_hbm,
                  sidx_v, didx_v, pk0_v, pk1_v, rows_v, acc, sem0, sem1):
    cid = lax.axis_index("c")
    sid = lax.axis_index("s")
    wid = cid * NS + sid
    rows_per_tile = NP // NS

    pltpu.sync_copy(zeros_hbm, acc.at[pl.ds(sid * rows_per_tile, rows_per_tile)])
    plsc.subcore_barrier()

    bufs = (pk0_v, pk1_v)
    sems = (sem0, sem1)

    def convert(pkb):
        def row(r, _):
            for k in range(4):
                w = pkb[r, pl.ds(16 * k, 16)]
                ab = plsc.bitcast(w, jnp.bfloat16)
                a, b = plsc.unpack(ab, format=plsc.PackFormat.INTERLEAVED)
                rows_v[r, pl.ds(32 * k, 16)] = a
                rows_v[r, pl.ds(32 * k + 16, 16)] = b
            return ()

        lax.fori_loop(0, CHUNK, row, (), unroll=8)

    for phase in range(CPT // H):
        pltpu.sync_copy(src_hbm.at[pl.ds(wid * CPT + phase * H, H)], sidx_v)
        pltpu.sync_copy(dst_hbm.at[pl.ds(wid * CPT + phase * H, H)], didx_v)
        pltpu.async_copy(g_hbm.at[sidx_v.at[0]], pk0_v, sem0)
        pltpu.async_copy(g_hbm.at[sidx_v.at[1]], pk1_v, sem1)

        def body(i, _):
            for b in range(2):
                t = 2 * i + b
                pltpu.make_async_copy(
                    g_hbm.at[sidx_v.at[t]], bufs[b], sems[b]).wait()
                convert(bufs[b])

                @pl.when(t + 2 < H)
                def _():
                    pltpu.async_copy(
                        g_hbm.at[sidx_v.at[t + 2]], bufs[b], sems[b])

                pltpu.sync_copy(rows_v, acc.at[didx_v.at[t]], add=True)
            return ()

        lax.fori_loop(0, H // 2, body, (), unroll=False)
    plsc.subcore_barrier()

    pltpu.sync_copy(
        acc.at[pl.ds(sid * rows_per_tile, rows_per_tile)],
        out_hbm.at[pl.ds(cid * NP + sid * rows_per_tile, rows_per_tile)],
    )


BLK = 512


def _dinv(d0, d1):
    return lax.rsqrt(d0 + d1 + 1.0)


def _hi_mask():
    return jnp.uint32(0xFFFF0000)


def _bf16_bits(g):
    return lax.bitcast_convert_type(
        g.astype(jnp.bfloat16).astype(jnp.float32), jnp.uint32)


def _pack(g_lo, g_hi):
    word = (_bf16_bits(g_hi) & _hi_mask()) | (_bf16_bits(g_lo) >> 16)
    return lax.bitcast_convert_type(word, jnp.float32)


def _unpack_halves(p):
    u = lax.bitcast_convert_type(p, jnp.uint32)
    g_lo = lax.bitcast_convert_type(u << 16, jnp.float32)
    g_hi = lax.bitcast_convert_type(u & _hi_mask(), jnp.float32)
    return g_lo, g_hi


def _scatter_mats():
    w = lax.broadcasted_iota(jnp.int32, (PD, D), 0)
    o = lax.broadcasted_iota(jnp.int32, (PD, D), 1)
    lo_col = 32 * (w // 16) + w % 16
    m_lo = (o == lo_col).astype(jnp.float32)
    m_hi = (o == lo_col + 16).astype(jnp.float32)
    return m_lo, m_hi


def _unpack(p):
    g_lo, g_hi = _unpack_halves(p)
    m_lo, m_hi = _scatter_mats()
    return (jnp.dot(g_lo, m_lo, preferred_element_type=jnp.float32)
            + jnp.dot(g_hi, m_hi, preferred_element_type=jnp.float32))


def _tc_lin_body(x_ref, wlo_ref, whi_ref, d0_ref, d1_ref, o_ref):
    dinv = _dinv(d0_ref[...], d1_ref[...])
    x = x_ref[...]
    g_lo = dinv * jnp.dot(x, wlo_ref[...], preferred_element_type=jnp.float32)
    g_hi = dinv * jnp.dot(x, whi_ref[...], preferred_element_type=jnp.float32)
    o_ref[...] = _pack(g_lo, g_hi)


def _tc_mid_body(p0_ref, p1_ref, g0_ref, d0_ref, d1_ref, b_ref,
                 wlo_ref, whi_ref, o_ref):
    dinv = _dinv(d0_ref[...], d1_ref[...])
    h = dinv * (p0_ref[...] + p1_ref[...] + _unpack(g0_ref[...])) + b_ref[...]
    h = jnp.maximum(h, 0.0)
    nrm = jnp.sqrt(jnp.sum(h * h, axis=1, keepdims=True))
    h = h / jnp.maximum(nrm, 1e-12)
    g_lo = dinv * jnp.dot(h, wlo_ref[...], preferred_element_type=jnp.float32)
    g_hi = dinv * jnp.dot(h, whi_ref[...], preferred_element_type=jnp.float32)
    o_ref[...] = _pack(g_lo, g_hi)


def _tc_fin_body(q0_ref, q1_ref, g1_ref, d0_ref, d1_ref, b_ref, o_ref):
    dinv = _dinv(d0_ref[...], d1_ref[...])
    o_ref[...] = (dinv * (q0_ref[...] + q1_ref[...] + _unpack(g1_ref[...]))
                  + b_ref[...])


def _row_spec(width=D):
    return pl.BlockSpec((BLK, width), lambda i: (i, 0))


def _col_spec():
    return pl.BlockSpec((BLK, 1), lambda i: (i, 0))


def _full_spec(shape):
    return pl.BlockSpec(shape, lambda i: (0,) * len(shape))


_GRID = (NP // BLK,)
_ROW_OUT = jax.ShapeDtypeStruct((NP, D), jnp.float32)
_PK_OUT = jax.ShapeDtypeStruct((NP, PD), jnp.float32)


def _tc_lin(x, wlo, whi, d0, d1):
    return pl.pallas_call(
        _tc_lin_body,
        grid=_GRID,
        in_specs=[_row_spec(), _full_spec((D, PD)), _full_spec((D, PD)),
                  _col_spec(), _col_spec()],
        out_specs=_row_spec(PD),
        out_shape=_PK_OUT,
    )(x, wlo, whi, d0, d1)


def _tc_mid(p0, p1, g0, d0, d1, b, wlo, whi):
    return pl.pallas_call(
        _tc_mid_body,
        grid=_GRID,
        in_specs=[_row_spec(), _row_spec(), _row_spec(PD), _col_spec(),
                  _col_spec(), _full_spec((1, D)), _full_spec((D, PD)),
                  _full_spec((D, PD))],
        out_specs=_row_spec(PD),
        out_shape=_PK_OUT,
    )(p0, p1, g0, d0, d1, b, wlo, whi)


def _tc_fin(q0, q1, g1, d0, d1, b):
    return pl.pallas_call(
        _tc_fin_body,
        grid=_GRID,
        in_specs=[_row_spec(), _row_spec(), _row_spec(PD), _col_spec(),
                  _col_spec(), _full_spec((1, D))],
        out_specs=_row_spec(),
        out_shape=_ROW_OUT,
    )(q0, q1, g1, d0, d1, b)


@jax.jit
def kernel(x, edge_index, W0, b0, W1, b1):
    src = edge_index[0]
    dst = edge_index[1]
    npad = E_PAD - E
    pad_iota = jnp.arange(npad, dtype=jnp.int32)
    pad_dst = N + pad_iota % (NP - N)
    pad_src = pad_iota % N
    src_p = jnp.concatenate([src, pad_src])
    dst_p = jnp.concatenate([dst, pad_dst])
    src2d = src_p.reshape(NW * CPT, CHUNK)
    dst2d = dst_p.reshape(NW * CPT, CHUNK)

    x_pad = jnp.concatenate([x, jnp.zeros((NP - N, D), jnp.float32)])
    zeros_np = jnp.zeros((NP,), jnp.float32)
    ones_chunk = jnp.ones((CHUNK,), jnp.float32)
    zeros_rows = jnp.zeros((NP // NS, D), jnp.float32)

    deg2 = _sc_degree(dst2d, zeros_np, ones_chunk)
    d0 = deg2[:NP].reshape(NP, 1)
    d1 = deg2[NP:].reshape(NP, 1)

    b0_2d = b0.reshape(1, D)
    b1_2d = b1.reshape(1, D)
    w0lo, w0hi = W0[:, LO_COLS], W0[:, LO_COLS + 16]
    w1lo, w1hi = W1[:, LO_COLS], W1[:, LO_COLS + 16]

    g0 = _tc_lin(x_pad, w0lo, w0hi, d0, d1)
    parts0 = _sc_aggregate(g0, src2d, dst2d, zeros_rows)
    g1 = _tc_mid(parts0[:NP], parts0[NP:], g0, d0, d1, b0_2d, w1lo, w1hi)
    parts1 = _sc_aggregate(g1, src2d, dst2d, zeros_rows)
    out = _tc_fin(parts1[:NP], parts1[NP:], g1, d0, d1, b1_2d)
    return out[:N]


# speedup vs baseline: 1.8991x; 1.8930x over previous
import functools

import jax
import jax.numpy as jnp
from jax import lax
from jax.experimental import pallas as pl
from jax.experimental.pallas import tpu as pltpu
from jax.experimental.pallas import tpu_sc as plsc

N = 10000
E = 320000
D = 128

NC = 2
NS = 16
NW = NC * NS

NP = 10240
CHUNK = 128
CPT = 80
EPT = CPT * CHUNK
E_PAD = NW * EPT
H = 40
CPT_C0 = 80
CPT_C1 = 80

_MESH = plsc.VectorSubcoreMesh(core_axis_name="c", subcore_axis_name="s")


@functools.partial(
    pl.kernel,
    out_type=jax.ShapeDtypeStruct((NC * NP,), jnp.float32),
    mesh=_MESH,
    scratch_types=[
        pltpu.VMEM((CPT, CHUNK), jnp.int32),
        pltpu.VMEM((CHUNK,), jnp.float32),
        pltpu.VMEM_SHARED((NP,), jnp.float32),
    ],
)
def _sc_degree(dst_hbm, zeros_np_hbm, ones_hbm, out_hbm, didx_v, ones_v, acc):
    cid = lax.axis_index("c")
    sid = lax.axis_index("s")
    wid = cid * NS + sid

    @pl.when(sid == 0)
    def _():
        pltpu.sync_copy(zeros_np_hbm, acc)

    pltpu.sync_copy(dst_hbm.at[pl.ds(wid * CPT, CPT)], didx_v)
    pltpu.sync_copy(ones_hbm, ones_v)
    plsc.subcore_barrier()

    def body(t, _):
        pltpu.sync_copy(ones_v, acc.at[didx_v.at[t]], add=True)
        return ()

    lax.fori_loop(0, CPT, body, (), unroll=False)
    plsc.subcore_barrier()

    @pl.when(sid == 0)
    def _():
        pltpu.sync_copy(acc, out_hbm.at[pl.ds(cid * NP, NP)])


@functools.partial(
    pl.kernel,
    out_type=jax.ShapeDtypeStruct((NC * NP, D), jnp.float32),
    mesh=_MESH,
    scratch_types=[
        pltpu.VMEM((H, CHUNK), jnp.int32),
        pltpu.VMEM((H, CHUNK), jnp.int32),
        pltpu.VMEM((CHUNK, D), jnp.float32),
        pltpu.VMEM((CHUNK, D), jnp.float32),
        pltpu.VMEM_SHARED((NP, D), jnp.float32),
        pltpu.SemaphoreType.DMA,
        pltpu.SemaphoreType.DMA,
    ],
)
def _sc_aggregate(g_hbm, src_hbm, dst_hbm, zeros_hbm, out_hbm,
                  sidx_v, didx_v, rows0_v, rows1_v, acc, sem0, sem1):
    cid = lax.axis_index("c")
    sid = lax.axis_index("s")
    rows_per_tile = NP // NS

    pltpu.sync_copy(zeros_hbm, acc.at[pl.ds(sid * rows_per_tile, rows_per_tile)])
    plsc.subcore_barrier()

    bufs = (rows0_v, rows1_v)
    sems = (sem0, sem1)

    def run(base, nchunks):
        for phase in range(nchunks // H):
            pltpu.sync_copy(src_hbm.at[pl.ds(base + phase * H, H)], sidx_v)
            pltpu.sync_copy(dst_hbm.at[pl.ds(base + phase * H, H)], didx_v)
            pltpu.async_copy(g_hbm.at[sidx_v.at[0]], rows0_v, sem0)
            pltpu.async_copy(g_hbm.at[sidx_v.at[1]], rows1_v, sem1)

            def body(i, _):
                for b in range(2):
                    t = 2 * i + b
                    pltpu.make_async_copy(
                        g_hbm.at[sidx_v.at[t]], bufs[b], sems[b]).wait()
                    pltpu.sync_copy(bufs[b], acc.at[didx_v.at[t]], add=True)

                    @pl.when(t + 2 < H)
                    def _():
                        pltpu.async_copy(
                            g_hbm.at[sidx_v.at[t + 2]], bufs[b], sems[b])
                return ()

            lax.fori_loop(0, H // 2, body, (), unroll=False)

    @pl.when(cid == 0)
    def _():
        run(sid * (CPT_C0 + CPT_C1), CPT_C0)

    @pl.when(cid == 1)
    def _():
        run(sid * (CPT_C0 + CPT_C1) + CPT_C0, CPT_C1)

    plsc.subcore_barrier()

    pltpu.sync_copy(
        acc.at[pl.ds(sid * rows_per_tile, rows_per_tile)],
        out_hbm.at[pl.ds(cid * NP + sid * rows_per_tile, rows_per_tile)],
    )


BLK = 512


def _dinv(d0, d1):
    return lax.rsqrt(d0 + d1 + 1.0)


def _tc_lin_body(x_ref, w_ref, d0_ref, d1_ref, o_ref):
    dinv = _dinv(d0_ref[...], d1_ref[...])
    o_ref[...] = dinv * jnp.dot(x_ref[...], w_ref[...],
                                preferred_element_type=jnp.float32)


def _tc_mid_body(p0_ref, p1_ref, g0_ref, d0_ref, d1_ref, b_ref, w_ref, o_ref):
    dinv = _dinv(d0_ref[...], d1_ref[...])
    h = dinv * (p0_ref[...] + p1_ref[...] + g0_ref[...]) + b_ref[...]
    h = jnp.maximum(h, 0.0)
    nrm = jnp.sqrt(jnp.sum(h * h, axis=1, keepdims=True))
    h = h / jnp.maximum(nrm, 1e-12)
    o_ref[...] = dinv * jnp.dot(h, w_ref[...],
                                preferred_element_type=jnp.float32)


def _tc_fin_body(q0_ref, q1_ref, g1_ref, d0_ref, d1_ref, b_ref, o_ref):
    dinv = _dinv(d0_ref[...], d1_ref[...])
    o_ref[...] = dinv * (q0_ref[...] + q1_ref[...] + g1_ref[...]) + b_ref[...]


def _row_spec():
    return pl.BlockSpec((BLK, D), lambda i: (i, 0))


def _col_spec():
    return pl.BlockSpec((BLK, 1), lambda i: (i, 0))


def _full_spec(shape):
    return pl.BlockSpec(shape, lambda i: (0,) * len(shape))


_GRID = (NP // BLK,)
_ROW_OUT = jax.ShapeDtypeStruct((NP, D), jnp.float32)


def _tc_lin(x, w, d0, d1):
    return pl.pallas_call(
        _tc_lin_body,
        grid=_GRID,
        in_specs=[_row_spec(), _full_spec((D, D)), _col_spec(), _col_spec()],
        out_specs=_row_spec(),
        out_shape=_ROW_OUT,
    )(x, w, d0, d1)


def _tc_mid(p0, p1, g0, d0, d1, b, w):
    return pl.pallas_call(
        _tc_mid_body,
        grid=_GRID,
        in_specs=[_row_spec(), _row_spec(), _row_spec(), _col_spec(),
                  _col_spec(), _full_spec((1, D)), _full_spec((D, D))],
        out_specs=_row_spec(),
        out_shape=_ROW_OUT,
    )(p0, p1, g0, d0, d1, b, w)


def _tc_fin(q0, q1, g1, d0, d1, b):
    return pl.pallas_call(
        _tc_fin_body,
        grid=_GRID,
        in_specs=[_row_spec(), _row_spec(), _row_spec(), _col_spec(),
                  _col_spec(), _full_spec((1, D))],
        out_specs=_row_spec(),
        out_shape=_ROW_OUT,
    )(q0, q1, g1, d0, d1, b)


@jax.jit
def kernel(x, edge_index, W0, b0, W1, b1):
    src = edge_index[0]
    dst = edge_index[1]
    npad = E_PAD - E
    pad_iota = jnp.arange(npad, dtype=jnp.int32)
    pad_dst = N + pad_iota % (NP - N)
    pad_src = pad_iota % N
    src_p = jnp.concatenate([src, pad_src])
    dst_p = jnp.concatenate([dst, pad_dst])
    src2d = src_p.reshape(NW * CPT, CHUNK)
    dst2d = dst_p.reshape(NW * CPT, CHUNK)

    x_pad = jnp.concatenate([x, jnp.zeros((NP - N, D), jnp.float32)])
    zeros_np = jnp.zeros((NP,), jnp.float32)
    ones_chunk = jnp.ones((CHUNK,), jnp.float32)
    zeros_rows = jnp.zeros((NP // NS, D), jnp.float32)

    deg2 = _sc_degree(dst2d, zeros_np, ones_chunk)
    d0 = deg2[:NP].reshape(NP, 1)
    d1 = deg2[NP:].reshape(NP, 1)

    b0_2d = b0.reshape(1, D)
    b1_2d = b1.reshape(1, D)

    g0 = _tc_lin(x_pad, W0, d0, d1)
    parts0 = _sc_aggregate(g0, src2d, dst2d, zeros_rows)
    g1 = _tc_mid(parts0[:NP], parts0[NP:], g0, d0, d1, b0_2d, W1)
    parts1 = _sc_aggregate(g1, src2d, dst2d, zeros_rows)
    out = _tc_fin(parts1[:NP], parts1[NP:], g1, d0, d1, b1_2d)
    return out[:N]


# speedup vs baseline: 1.9204x; 1.0112x over previous
import functools

import jax
import jax.numpy as jnp
from jax import lax
from jax.experimental import pallas as pl
from jax.experimental.pallas import tpu as pltpu
from jax.experimental.pallas import tpu_sc as plsc

N = 10000
E = 320000
D = 128

NC = 2
NS = 16
NW = NC * NS

NP = 10240
CHUNK = 128
CPT = 80
EPT = CPT * CHUNK
E_PAD = NW * EPT
H = 40
CPT_C0 = 80
CPT_C1 = 80

_MESH = plsc.VectorSubcoreMesh(core_axis_name="c", subcore_axis_name="s")


@functools.partial(
    pl.kernel,
    out_type=jax.ShapeDtypeStruct((NC * NP,), jnp.float32),
    mesh=_MESH,
    scratch_types=[
        pltpu.VMEM((CPT, CHUNK), jnp.int32),
        pltpu.VMEM((CHUNK,), jnp.float32),
        pltpu.VMEM_SHARED((NP,), jnp.float32),
    ],
)
def _sc_degree(dst_hbm, zeros_np_hbm, ones_hbm, out_hbm, didx_v, ones_v, acc):
    cid = lax.axis_index("c")
    sid = lax.axis_index("s")
    wid = cid * NS + sid

    @pl.when(sid == 0)
    def _():
        pltpu.sync_copy(zeros_np_hbm, acc)

    pltpu.sync_copy(dst_hbm.at[pl.ds(wid * CPT, CPT)], didx_v)
    pltpu.sync_copy(ones_hbm, ones_v)
    plsc.subcore_barrier()

    def body(t, _):
        pltpu.sync_copy(ones_v, acc.at[didx_v.at[t]], add=True)
        return ()

    lax.fori_loop(0, CPT, body, (), unroll=False)
    plsc.subcore_barrier()

    @pl.when(sid == 0)
    def _():
        pltpu.sync_copy(acc, out_hbm.at[pl.ds(cid * NP, NP)])


@functools.partial(
    pl.kernel,
    out_type=jax.ShapeDtypeStruct((NC * NP, D), jnp.float32),
    mesh=_MESH,
    scratch_types=[
        pltpu.VMEM((H, CHUNK), jnp.int32),
        pltpu.VMEM((H, CHUNK), jnp.int32),
        pltpu.VMEM((CHUNK, D), jnp.float32),
        pltpu.VMEM((CHUNK, D), jnp.float32),
        pltpu.VMEM_SHARED((NP, D), jnp.float32),
        pltpu.SemaphoreType.DMA,
        pltpu.SemaphoreType.DMA,
        pltpu.SemaphoreType.DMA,
    ],
)
def _sc_aggregate(g_hbm, src_hbm, dst_hbm, zeros_hbm, out_hbm,
                  sidx_v, didx_v, rows0_v, rows1_v, acc, sem0, sem1, semz):
    cid = lax.axis_index("c")
    sid = lax.axis_index("s")
    rows_per_tile = NP // NS
    zero_dst = acc.at[pl.ds(sid * rows_per_tile, rows_per_tile)]

    pltpu.async_copy(zeros_hbm, zero_dst, semz)

    bufs = (rows0_v, rows1_v)
    sems = (sem0, sem1)

    def run(base, nchunks):
        for phase in range(nchunks // H):
            pltpu.sync_copy(src_hbm.at[pl.ds(base + phase * H, H)], sidx_v)
            pltpu.sync_copy(dst_hbm.at[pl.ds(base + phase * H, H)], didx_v)
            pltpu.async_copy(g_hbm.at[sidx_v.at[0]], rows0_v, sem0)
            pltpu.async_copy(g_hbm.at[sidx_v.at[1]], rows1_v, sem1)
            if phase == 0:
                pltpu.make_async_copy(zeros_hbm, zero_dst, semz).wait()
                plsc.subcore_barrier()

            def body(i, _):
                for b in range(2):
                    t = 2 * i + b
                    pltpu.make_async_copy(
                        g_hbm.at[sidx_v.at[t]], bufs[b], sems[b]).wait()
                    pltpu.sync_copy(bufs[b], acc.at[didx_v.at[t]], add=True)

                    @pl.when(t + 2 < H)
                    def _():
                        pltpu.async_copy(
                            g_hbm.at[sidx_v.at[t + 2]], bufs[b], sems[b])
                return ()

            lax.fori_loop(0, H // 2, body, (), unroll=False)

    @pl.when(cid == 0)
    def _():
        run(sid * (CPT_C0 + CPT_C1), CPT_C0)

    @pl.when(cid == 1)
    def _():
        run(sid * (CPT_C0 + CPT_C1) + CPT_C0, CPT_C1)

    plsc.subcore_barrier()

    pltpu.sync_copy(
        acc.at[pl.ds(sid * rows_per_tile, rows_per_tile)],
        out_hbm.at[pl.ds(cid * NP + sid * rows_per_tile, rows_per_tile)],
    )


BLK = 512


def _dinv(d0, d1):
    return lax.rsqrt(d0 + d1 + 1.0)


def _tc_lin_body(x_ref, w_ref, d0_ref, d1_ref, o_ref):
    dinv = _dinv(d0_ref[...], d1_ref[...])
    o_ref[...] = dinv * jnp.dot(x_ref[...], w_ref[...],
                                preferred_element_type=jnp.float32)


def _tc_mid_body(p0_ref, p1_ref, g0_ref, d0_ref, d1_ref, b_ref, w_ref, o_ref):
    dinv = _dinv(d0_ref[...], d1_ref[...])
    h = dinv * (p0_ref[...] + p1_ref[...] + g0_ref[...]) + b_ref[...]
    h = jnp.maximum(h, 0.0)
    nrm = jnp.sqrt(jnp.sum(h * h, axis=1, keepdims=True))
    h = h / jnp.maximum(nrm, 1e-12)
    o_ref[...] = dinv * jnp.dot(h, w_ref[...],
                                preferred_element_type=jnp.float32)


def _tc_fin_body(q0_ref, q1_ref, g1_ref, d0_ref, d1_ref, b_ref, o_ref):
    dinv = _dinv(d0_ref[...], d1_ref[...])
    o_ref[...] = dinv * (q0_ref[...] + q1_ref[...] + g1_ref[...]) + b_ref[...]


def _row_spec():
    return pl.BlockSpec((BLK, D), lambda i: (i, 0))


def _col_spec():
    return pl.BlockSpec((BLK, 1), lambda i: (i, 0))


def _full_spec(shape):
    return pl.BlockSpec(shape, lambda i: (0,) * len(shape))


_GRID = (NP // BLK,)
_ROW_OUT = jax.ShapeDtypeStruct((NP, D), jnp.float32)


def _tc_lin(x, w, d0, d1):
    return pl.pallas_call(
        _tc_lin_body,
        grid=_GRID,
        in_specs=[_row_spec(), _full_spec((D, D)), _col_spec(), _col_spec()],
        out_specs=_row_spec(),
        out_shape=_ROW_OUT,
    )(x, w, d0, d1)


def _tc_mid(p0, p1, g0, d0, d1, b, w):
    return pl.pallas_call(
        _tc_mid_body,
        grid=_GRID,
        in_specs=[_row_spec(), _row_spec(), _row_spec(), _col_spec(),
                  _col_spec(), _full_spec((1, D)), _full_spec((D, D))],
        out_specs=_row_spec(),
        out_shape=_ROW_OUT,
    )(p0, p1, g0, d0, d1, b, w)


def _tc_fin(q0, q1, g1, d0, d1, b):
    return pl.pallas_call(
        _tc_fin_body,
        grid=_GRID,
        in_specs=[_row_spec(), _row_spec(), _row_spec(), _col_spec(),
                  _col_spec(), _full_spec((1, D))],
        out_specs=_row_spec(),
        out_shape=_ROW_OUT,
    )(q0, q1, g1, d0, d1, b)


@jax.jit
def kernel(x, edge_index, W0, b0, W1, b1):
    src = edge_index[0]
    dst = edge_index[1]
    npad = E_PAD - E
    pad_iota = jnp.arange(npad, dtype=jnp.int32)
    pad_dst = N + pad_iota % (NP - N)
    pad_src = pad_iota % N
    src_p = jnp.concatenate([src, pad_src])
    dst_p = jnp.concatenate([dst, pad_dst])
    src2d = src_p.reshape(NW * CPT, CHUNK)
    dst2d = dst_p.reshape(NW * CPT, CHUNK)

    x_pad = jnp.concatenate([x, jnp.zeros((NP - N, D), jnp.float32)])
    zeros_np = jnp.zeros((NP,), jnp.float32)
    ones_chunk = jnp.ones((CHUNK,), jnp.float32)
    zeros_rows = jnp.zeros((NP // NS, D), jnp.float32)

    deg2 = _sc_degree(dst2d, zeros_np, ones_chunk)
    d0 = deg2[:NP].reshape(NP, 1)
    d1 = deg2[NP:].reshape(NP, 1)

    b0_2d = b0.reshape(1, D)
    b1_2d = b1.reshape(1, D)

    g0 = _tc_lin(x_pad, W0, d0, d1)
    parts0 = _sc_aggregate(g0, src2d, dst2d, zeros_rows)
    g1 = _tc_mid(parts0[:NP], parts0[NP:], g0, d0, d1, b0_2d, W1)
    parts1 = _sc_aggregate(g1, src2d, dst2d, zeros_rows)
    out = _tc_fin(parts1[:NP], parts1[NP:], g1, d0, d1, b1_2d)
    return out[:N]
